# baseline (speedup 1.0000x reference)
import functools

import jax
import jax.numpy as jnp
from jax import lax
from jax.experimental import pallas as pl
from jax.experimental.pallas import tpu as pltpu
from jax.experimental.pallas import tpu_sc as plsc

_EMB_LEVELS = 1000000
_EMB_DIM = 16
_L = 16
_SEG = 128
_CHUNK = 256

_C1 = -1640531535
_C2 = 0x45D9F3B


def _hash_vec(xv):
    h = xv * jnp.int32(_C1)
    h = h ^ lax.shift_right_logical(h, 16)
    h = h * jnp.int32(_C2)
    h = h ^ lax.shift_right_logical(h, 16)
    q = lax.shift_right_logical(h, 1)
    r0 = h & jnp.int32(1)
    m = jnp.int32(_EMB_LEVELS)
    return lax.rem(lax.rem(q, m) * jnp.int32(2) + r0, m)


def _body(n_chunks, x_hbm, seeds_hbm, table_hbm, out_hbm,
          seeds_v, x_v, blk0_v, blk1_v, r0_v, r1_v, blkbuf, rows_v,
          gsem, osem):
    info = plsc.get_sparse_core_info()
    nc = info.num_cores
    wid = lax.axis_index("s") * nc + lax.axis_index("c")
    per_w = _CHUNK * n_chunks

    pltpu.sync_copy(seeds_hbm, seeds_v)
    s0 = seeds_v[0, :]
    s1 = seeds_v[1, :]
    m = jnp.int32(_EMB_LEVELS)
    blk_refs = [blk0_v, blk1_v]
    r_refs = [r0_v, r1_v]
    n_seg = _CHUNK // _SEG

    def do_chunk(c, _):
        base = wid * per_w + c * _CHUNK
        pltpu.sync_copy(x_hbm.at[pl.ds(base, _CHUNK)], x_v)

        def grp(g, _):
            xv = x_v[pl.ds(g * _L, _L)]
            h = _hash_vec(xv)
            for s_vec, blk_r, r_r in ((s0, blk0_v, r0_v), (s1, blk1_v, r1_v)):
                idx = lax.rem(h + s_vec, m)
                blk_r[pl.ds(g * _L, _L)] = lax.shift_right_logical(idx, 3)
                r_r[pl.ds(g * _L, _L)] = (idx & jnp.int32(7)) * jnp.int32(_EMB_DIM)
            return _

        lax.fori_loop(0, _CHUNK // _L, grp, None, unroll=2)

        def fire(t):
            s, seg = t // n_seg, t % n_seg
            return pltpu.async_copy(
                table_hbm.at[blk_refs[s].at[pl.ds(seg * _SEG, _SEG)]],
                blkbuf.at[t & 1], gsem)

        def extract(t):
            s, seg = t // n_seg, t % n_seg
            slot = t & 1
            r_r = r_refs[s]

            def egrp(g, _):
                rv = r_r[pl.ds(seg * _SEG + g * _L, _L)]
                rowbase = seg * _SEG + g * _L
                for j in range(_L):
                    o = rv[j]
                    rows_v[s, rowbase + j, :] = blkbuf[slot, g * _L + j, pl.ds(o, _EMB_DIM)]
                return _

            lax.fori_loop(0, _SEG // _L, egrp, None)

        handles = {}
        write_handles = []
        handles[0] = fire(0)
        for t in range(1, 2 * n_seg):
            handles[t] = fire(t)
            handles.pop(t - 1).wait()
            extract(t - 1)
            if t == n_seg:
                write_handles.append(pltpu.async_copy(
                    rows_v.at[0], out_hbm.at[pl.ds(base, _CHUNK), 0], osem))
        t = 2 * n_seg - 1
        handles.pop(t).wait()
        extract(t)
        write_handles.append(pltpu.async_copy(
            rows_v.at[1], out_hbm.at[pl.ds(base, _CHUNK), 1], osem))
        for h in write_handles:
            h.wait()
        return _

    lax.fori_loop(0, n_chunks, do_chunk, None)


def kernel(x, fnum, table):
    batch, fields = x.shape
    n = batch * fields
    x_flat = x.reshape(n)
    table_blk = table.reshape(_EMB_LEVELS // 8, 8 * _EMB_DIM)
    seeds = jnp.broadcast_to(fnum.reshape(2, 1), (2, _L)).astype(jnp.int32)

    info = plsc.get_sparse_core_info()
    nw = info.num_cores * info.num_subcores
    per_w = n // nw
    assert per_w * nw == n
    n_chunks = per_w // _CHUNK
    assert n_chunks * _CHUNK == per_w

    mesh = plsc.VectorSubcoreMesh(core_axis_name="c", subcore_axis_name="s")
    kfn = pl.kernel(
        functools.partial(_body, n_chunks),
        out_type=jax.ShapeDtypeStruct((n, 2, _EMB_DIM), jnp.float32),
        mesh=mesh,
        scratch_types=[
            pltpu.VMEM((2, _L), jnp.int32),
            pltpu.VMEM((_CHUNK,), jnp.int32),
            pltpu.VMEM((_CHUNK,), jnp.int32),
            pltpu.VMEM((_CHUNK,), jnp.int32),
            pltpu.VMEM((_CHUNK,), jnp.int32),
            pltpu.VMEM((_CHUNK,), jnp.int32),
            pltpu.VMEM((2, _SEG, 8 * _EMB_DIM), jnp.float32),
            pltpu.VMEM((2, _CHUNK, _EMB_DIM), jnp.float32),
            pltpu.SemaphoreType.DMA,
            pltpu.SemaphoreType.DMA,
        ],
    )
    out = kfn(x_flat, seeds, table_blk)
    return out.reshape(batch, fields, 2 * _EMB_DIM)


# speedup vs baseline: 2.4339x; 2.4339x over previous
import functools

import jax
import jax.numpy as jnp
from jax import lax
from jax.experimental import pallas as pl
from jax.experimental.pallas import tpu as pltpu
from jax.experimental.pallas import tpu_sc as plsc

_EMB_LEVELS = 1000000
_EMB_DIM = 16
_L = 16
_SEG = 128

_C1 = -1640531535
_C2 = 0x45D9F3B


def _hash_vec(xv):
    h = xv * jnp.int32(_C1)
    h = h ^ lax.shift_right_logical(h, 16)
    h = h * jnp.int32(_C2)
    h = h ^ lax.shift_right_logical(h, 16)
    q = lax.shift_right_logical(h, 1)
    r0 = h & jnp.int32(1)
    m = jnp.int32(_EMB_LEVELS)
    return lax.rem(lax.rem(q, m) * jnp.int32(2) + r0, m)


def _body(chunk, n_chunks, x_hbm, seeds_hbm, table_hbm, out_hbm,
          seeds_v, x_a, x_b, idx_a, idx_b, rows_a, rows_b, gsem, osem):
    info = plsc.get_sparse_core_info()
    nc = info.num_cores
    wid = lax.axis_index("s") * nc + lax.axis_index("c")
    per_w = chunk * n_chunks
    n_seg = chunk // _SEG

    pltpu.sync_copy(seeds_hbm, seeds_v)
    s0 = seeds_v[0, :]
    s1 = seeds_v[1, :]
    m = jnp.int32(_EMB_LEVELS)
    x_refs = [x_a, x_b]
    idx_refs = [idx_a, idx_b]
    rows_refs = [rows_a, rows_b]

    def load_x(c):
        base = wid * per_w + c * chunk
        pltpu.sync_copy(x_hbm.at[pl.ds(base, chunk)], x_refs[c & 1])

    def hash_chunk(c):
        x_v, idx_v = x_refs[c & 1], idx_refs[c & 1]

        def grp(g, _):
            xv = x_v[pl.ds(g * _L, _L)]
            h = _hash_vec(xv)
            idx_v[0, pl.ds(g * _L, _L)] = lax.rem(h + s0, m)
            idx_v[1, pl.ds(g * _L, _L)] = lax.rem(h + s1, m)
            return _

        lax.fori_loop(0, chunk // _L, grp, None, unroll=4)

    def fire_gathers(c):
        idx_v, rows_v = idx_refs[c & 1], rows_refs[c & 1]
        handles = []
        for s in (0, 1):
            for k in range(n_seg):
                handles.append(pltpu.async_copy(
                    table_hbm.at[idx_v.at[s, pl.ds(k * _SEG, _SEG)]],
                    rows_v.at[s, pl.ds(k * _SEG, _SEG)], gsem))
        return handles

    def fire_writes(c):
        base = wid * per_w + c * chunk
        rows_v = rows_refs[c & 1]
        return [
            pltpu.async_copy(rows_v.at[s],
                             out_hbm.at[pl.ds(base, chunk), s], osem)
            for s in (0, 1)
        ]

    wh = {}
    load_x(0)
    hash_chunk(0)
    for c in range(n_chunks):
        if c >= 2:
            for h in wh.pop(c - 2):
                h.wait()
        gh = fire_gathers(c)
        if c + 1 < n_chunks:
            load_x(c + 1)
            hash_chunk(c + 1)
        for h in gh:
            h.wait()
        wh[c] = fire_writes(c)
    for hs in wh.values():
        for h in hs:
            h.wait()


def kernel(x, fnum, table):
    batch, fields = x.shape
    n = batch * fields
    x_flat = x.reshape(n)
    seeds = jnp.broadcast_to(fnum.reshape(2, 1), (2, _L)).astype(jnp.int32)

    info = plsc.get_sparse_core_info()
    nw = info.num_cores * info.num_subcores
    per_w = n // nw
    assert per_w * nw == n
    chunk = 1664
    n_chunks = per_w // chunk
    assert n_chunks * chunk == per_w

    mesh = plsc.VectorSubcoreMesh(core_axis_name="c", subcore_axis_name="s")
    kfn = pl.kernel(
        functools.partial(_body, chunk, n_chunks),
        out_type=jax.ShapeDtypeStruct((n, 2, _EMB_DIM), jnp.float32),
        mesh=mesh,
        compiler_params=pltpu.CompilerParams(use_tc_tiling_on_sc=False),
        scratch_types=[
            pltpu.VMEM((2, _L), jnp.int32),
            pltpu.VMEM((chunk,), jnp.int32),
            pltpu.VMEM((chunk,), jnp.int32),
            pltpu.VMEM((2, chunk), jnp.int32),
            pltpu.VMEM((2, chunk), jnp.int32),
            pltpu.VMEM((2, chunk, _EMB_DIM), jnp.float32),
            pltpu.VMEM((2, chunk, _EMB_DIM), jnp.float32),
            pltpu.SemaphoreType.DMA,
            pltpu.SemaphoreType.DMA,
        ],
    )
    out = kfn(x_flat, seeds, table)
    return out.reshape(batch, fields, 2 * _EMB_DIM)


# speedup vs baseline: 2.4782x; 1.0182x over previous
import functools

import jax
import jax.numpy as jnp
from jax import lax
from jax.experimental import pallas as pl
from jax.experimental.pallas import tpu as pltpu
from jax.experimental.pallas import tpu_sc as plsc

_EMB_LEVELS = 1000000
_EMB_DIM = 16
_L = 16
_SEG = 128

_C1 = -1640531535
_C2 = 0x45D9F3B


def _hash_vec(xv):
    h = xv * jnp.int32(_C1)
    h = h ^ lax.shift_right_logical(h, 16)
    h = h * jnp.int32(_C2)
    h = h ^ lax.shift_right_logical(h, 16)
    q = lax.shift_right_logical(h, 1)
    r0 = h & jnp.int32(1)
    m = jnp.int32(_EMB_LEVELS)
    return lax.rem(lax.rem(q, m) * jnp.int32(2) + r0, m)


def _body(chunk, n_chunks, x_hbm, seeds_hbm, table_hbm, out_hbm,
          seeds_v, x_a, x_b, idx_a, idx_b, rows_a, rows_b, gsem, osem):
    info = plsc.get_sparse_core_info()
    nc = info.num_cores
    wid = lax.axis_index("s") * nc + lax.axis_index("c")
    per_w = chunk * n_chunks
    n_seg = chunk // _SEG

    pltpu.sync_copy(seeds_hbm, seeds_v)
    s0 = seeds_v[0, :]
    s1 = seeds_v[1, :]
    m = jnp.int32(_EMB_LEVELS)
    x_refs = [x_a, x_b]
    idx_refs = [idx_a, idx_b]
    rows_refs = [rows_a, rows_b]

    def load_x(c):
        base = wid * per_w + c * chunk
        pltpu.sync_copy(x_hbm.at[pl.ds(base, chunk)], x_refs[c & 1])

    def hash_chunk(c):
        x_v, idx_v = x_refs[c & 1], idx_refs[c & 1]

        def grp(g, _):
            xv = x_v[pl.ds(g * _L, _L)]
            h = _hash_vec(xv)
            idx_v[0, pl.ds(g * _L, _L)] = lax.rem(h + s0, m)
            idx_v[1, pl.ds(g * _L, _L)] = lax.rem(h + s1, m)
            return _

        lax.fori_loop(0, chunk // _L, grp, None, unroll=4)

    def fire_gathers(c):
        idx_v, rows_v = idx_refs[c & 1], rows_refs[c & 1]
        return [
            pltpu.async_copy(table_hbm.at[idx_v.at[s]], rows_v.at[s], gsem)
            for s in (0, 1)
        ]

    def fire_writes(c):
        base = wid * per_w + c * chunk
        rows_v = rows_refs[c & 1]
        return [
            pltpu.async_copy(rows_v.at[s],
                             out_hbm.at[pl.ds(base, chunk), s], osem)
            for s in (0, 1)
        ]

    wh = {}
    load_x(0)
    hash_chunk(0)
    for c in range(n_chunks):
        if c >= 2:
            for h in wh.pop(c - 2):
                h.wait()
        gh = fire_gathers(c)
        if c + 1 < n_chunks:
            load_x(c + 1)
            hash_chunk(c + 1)
        for h in gh:
            h.wait()
        wh[c] = fire_writes(c)
    for hs in wh.values():
        for h in hs:
            h.wait()


def kernel(x, fnum, table):
    batch, fields = x.shape
    n = batch * fields
    x_flat = x.reshape(n)
    seeds = jnp.broadcast_to(fnum.reshape(2, 1), (2, _L)).astype(jnp.int32)

    info = plsc.get_sparse_core_info()
    nw = info.num_cores * info.num_subcores
    per_w = n // nw
    assert per_w * nw == n
    chunk = 1664
    n_chunks = per_w // chunk
    assert n_chunks * chunk == per_w

    mesh = plsc.VectorSubcoreMesh(core_axis_name="c", subcore_axis_name="s")
    kfn = pl.kernel(
        functools.partial(_body, chunk, n_chunks),
        out_type=jax.ShapeDtypeStruct((n, 2, _EMB_DIM), jnp.float32),
        mesh=mesh,
        compiler_params=pltpu.CompilerParams(use_tc_tiling_on_sc=False),
        scratch_types=[
            pltpu.VMEM((2, _L), jnp.int32),
            pltpu.VMEM((chunk,), jnp.int32),
            pltpu.VMEM((chunk,), jnp.int32),
            pltpu.VMEM((2, chunk), jnp.int32),
            pltpu.VMEM((2, chunk), jnp.int32),
            pltpu.VMEM((2, chunk, _EMB_DIM), jnp.float32),
            pltpu.VMEM((2, chunk, _EMB_DIM), jnp.float32),
            pltpu.SemaphoreType.DMA,
            pltpu.SemaphoreType.DMA,
        ],
    )
    out = kfn(x_flat, seeds, table)
    return out.reshape(batch, fields, 2 * _EMB_DIM)
